# Initial kernel scaffold; baseline (speedup 1.0000x reference)
import jax
import jax.numpy as jnp
from jax.experimental import pallas as pl


def kernel(Z, W):
    raise NotImplementedError("write your pallas kernel here")


# speedup vs baseline: 1.4757x; 1.4757x over previous
import functools

import jax
import jax.numpy as jnp
from jax import lax
from jax.experimental import pallas as pl
from jax.experimental.pallas import tpu as pltpu
from jax.experimental.pallas import tpu_sc as plsc

EMB = 128
NTYPES = 64
N = 100000
CHUNK = 80
NCHUNKS = N // CHUNK
NW = 32


def _body(w_hbm, z_hbm, out_hbm, idx_v, rows_v, sem):
    wid = lax.axis_index("s") * 2 + lax.axis_index("c")
    n_mine = (NCHUNKS - 1 - wid) // NW + 1

    def step(t, carry):
        base = (wid + t * NW) * CHUNK
        pltpu.sync_copy(z_hbm.at[pl.ds(base, CHUNK)], idx_v)
        for i in range(CHUNK // 16):
            sl = pl.ds(i * 16, 16)
            idx_v[sl] = idx_v[sl] - 1
        pltpu.async_copy(w_hbm.at[idx_v], rows_v, sem).wait()
        pltpu.sync_copy(rows_v, out_hbm.at[pl.ds(base, CHUNK)])
        return carry

    lax.fori_loop(0, n_mine, step, 0)


def kernel(Z, W):
    mesh = plsc.VectorSubcoreMesh(core_axis_name="c", subcore_axis_name="s")
    k = functools.partial(
        pl.kernel,
        mesh=mesh,
        out_type=jax.ShapeDtypeStruct((N, EMB), jnp.float32),
        scratch_types=[
            pltpu.VMEM((CHUNK,), jnp.int32),
            pltpu.VMEM((CHUNK, EMB), jnp.float32),
            pltpu.SemaphoreType.DMA,
        ],
    )(_body)
    return k(W, Z)


# speedup vs baseline: 1.5847x; 1.0739x over previous
import functools

import jax
import jax.numpy as jnp
from jax import lax
from jax.experimental import pallas as pl
from jax.experimental.pallas import tpu as pltpu
from jax.experimental.pallas import tpu_sc as plsc

EMB = 128
NTYPES = 64
N = 100000
CHUNK = 80
NCHUNKS = N // CHUNK
NW = 32
NB = 3
NOUTER = 39 // NB


def _body(w_hbm, z_hbm, out_hbm, *scratch):
    idx = scratch[0:NB]
    rows = scratch[NB:2 * NB]
    gsem = scratch[2 * NB:3 * NB]
    osem = scratch[3 * NB:4 * NB]
    wid = lax.axis_index("s") * 2 + lax.axis_index("c")

    def chunk_base(t):
        return (wid + t * NW) * CHUNK

    def load_and_gather(t, b):
        pltpu.sync_copy(z_hbm.at[pl.ds(chunk_base(t), CHUNK)], idx[b])
        for i in range(CHUNK // 16):
            sl = pl.ds(i * 16, 16)
            idx[b][sl] = idx[b][sl] - 1
        pltpu.make_async_copy(w_hbm.at[idx[b]], rows[b], gsem[b]).start()

    def out_desc(t, b):
        return pltpu.make_async_copy(
            rows[b], out_hbm.at[pl.ds(chunk_base(t), CHUNK)], osem[b])

    def finish_chunk(t, b):
        pltpu.make_async_copy(w_hbm.at[idx[b]], rows[b], gsem[b]).wait()
        out_desc(t, b).start()

    for b in range(NB):
        load_and_gather(b, b)

    def outer(g, carry):
        t0 = g * NB
        for b in range(NB):
            finish_chunk(t0 + b, b)
        @pl.when(g + 1 < NOUTER)
        def _():
            for b in range(NB):
                out_desc(t0 + b, b).wait()
                load_and_gather(t0 + NB + b, b)
        return carry

    lax.fori_loop(0, NOUTER, outer, 0)

    for b in range(NB):
        out_desc((NOUTER - 1) * NB + b, b).wait()

    @pl.when(wid < NCHUNKS - NOUTER * NB * NW)
    def _():
        load_and_gather(NOUTER * NB, 0)
        finish_chunk(NOUTER * NB, 0)
        out_desc(NOUTER * NB, 0).wait()


def kernel(Z, W):
    mesh = plsc.VectorSubcoreMesh(core_axis_name="c", subcore_axis_name="s")
    k = functools.partial(
        pl.kernel,
        mesh=mesh,
        out_type=jax.ShapeDtypeStruct((N, EMB), jnp.float32),
        scratch_types=(
            [pltpu.VMEM((CHUNK,), jnp.int32) for _ in range(NB)]
            + [pltpu.VMEM((CHUNK, EMB), jnp.float32) for _ in range(NB)]
            + [pltpu.SemaphoreType.DMA for _ in range(2 * NB)]
        ),
    )(_body)
    return k(W, Z)
